# baseline (speedup 1.0000x reference)
import functools
import jax
import jax.numpy as jnp
from jax import lax
from jax.experimental import pallas as pl
from jax.experimental.pallas import tpu as pltpu, tpu_sc as plsc

_D = 128
_CHUNK = 256
_NBUF = 3
_NQ = 16
_NP = 2


def _make_gather(batch, hist):
    n_chunks = hist // _NP
    cols = batch // _NQ
    assert cols == _CHUNK
    n_outer = (n_chunks - 1) // _NBUF
    assert n_outer * _NBUF + 1 == n_chunks
    mesh = plsc.VectorSubcoreMesh(core_axis_name="c", subcore_axis_name="s")

    scratch = (
        [pltpu.VMEM((hist, _CHUNK), jnp.int32)]
        + [pltpu.VMEM((_CHUNK, _D), jnp.float32) for _ in range(_NBUF)]
        + [pltpu.SemaphoreType.DMA for _ in range(2 * _NBUF)]
    )

    @functools.partial(
        pl.kernel,
        out_type=jax.ShapeDtypeStruct((hist * batch, _D), jnp.float32),
        mesh=mesh,
        scratch_types=scratch,
    )
    def gather_kernel(ids_hbm, table_hbm, out_hbm, idx_v, *rest):
        bufs = rest[:_NBUF]
        g_sems = rest[_NBUF : 2 * _NBUF]
        o_sems = rest[2 * _NBUF :]
        wid = lax.axis_index("s") * 2 + lax.axis_index("c")
        q = wid // _NP
        p = wid % _NP
        h0 = p * n_chunks
        col0 = q * cols
        pltpu.sync_copy(ids_hbm.at[:, pl.ds(col0, cols)], idx_v)

        def g_start(j, b):
            for k in range(_CHUNK // _D):
                pltpu.async_copy(
                    table_hbm.at[idx_v.at[h0 + j, pl.ds(k * _D, _D)]],
                    bufs[b].at[pl.ds(k * _D, _D)],
                    g_sems[b],
                )

        def g_wait(b):
            pltpu.make_async_copy(
                table_hbm.at[idx_v.at[0, pl.ds(0, _D)]], bufs[b], g_sems[b]
            ).wait()

        def o_start(j, b):
            pltpu.async_copy(
                bufs[b],
                out_hbm.at[pl.ds((h0 + j) * batch + col0, _CHUNK)],
                o_sems[b],
            )

        def o_wait(b):
            pltpu.make_async_copy(
                bufs[b], out_hbm.at[pl.ds(col0, _CHUNK)], o_sems[b]
            ).wait()

        for b in range(_NBUF - 1):
            g_start(b, b)

        def outer(o, carry):
            for b in range(_NBUF):
                j = o * _NBUF + b
                g_wait(b)
                o_start(j, b)
                pb = (b - 1) % _NBUF
                nxt = j + _NBUF - 1
                if b == 0:
                    @pl.when(o > 0)
                    def _():
                        o_wait(pb)

                    g_start(nxt, pb)
                elif (b + _NBUF - 1) <= _NBUF:
                    o_wait(pb)
                    g_start(nxt, pb)
                else:
                    o_wait(pb)

                    @pl.when(o < n_outer - 1)
                    def _():
                        g_start(nxt, pb)

            return carry

        lax.fori_loop(0, n_outer, outer, 0)

        g_wait(0)
        o_start(n_chunks - 1, 0)

        o_wait(_NBUF - 1)
        o_wait(0)

    return gather_kernel


def kernel(ids, table):
    batch, hist = ids.shape
    ids_t = ids.T.astype(jnp.int32)
    out = _make_gather(batch, hist)(ids_t, table)
    return out.reshape(hist, batch, _D).transpose(1, 0, 2)


# speedup vs baseline: 1.0290x; 1.0290x over previous
import functools
import jax
import jax.numpy as jnp
from jax import lax
from jax.experimental import pallas as pl
from jax.experimental.pallas import tpu as pltpu, tpu_sc as plsc

_D = 128
_NW = 32
_CHUNK = 128
_NBUF = 7


def _make_gather(batch, hist):
    n_chunks = hist
    cols = batch // _NW
    assert cols == _CHUNK
    n_outer = (n_chunks - 1) // _NBUF
    assert n_outer * _NBUF + 1 == n_chunks
    mesh = plsc.VectorSubcoreMesh(core_axis_name="c", subcore_axis_name="s")

    scratch = (
        [pltpu.VMEM((hist, cols), jnp.int32)]
        + [pltpu.VMEM((_CHUNK, _D), jnp.float32) for _ in range(_NBUF)]
        + [pltpu.SemaphoreType.DMA for _ in range(2 * _NBUF)]
    )

    @functools.partial(
        pl.kernel,
        out_type=jax.ShapeDtypeStruct((hist * batch, _D), jnp.float32),
        mesh=mesh,
        scratch_types=scratch,
    )
    def gather_kernel(ids_hbm, table_hbm, out_hbm, idx_v, *rest):
        bufs = rest[:_NBUF]
        g_sems = rest[_NBUF : 2 * _NBUF]
        o_sems = rest[2 * _NBUF :]
        wid = lax.axis_index("s") * 2 + lax.axis_index("c")
        col0 = wid * cols
        pltpu.sync_copy(ids_hbm.at[:, pl.ds(col0, cols)], idx_v)

        def g_start(h, b):
            pltpu.async_copy(table_hbm.at[idx_v.at[h]], bufs[b], g_sems[b])

        def g_wait(b):
            pltpu.make_async_copy(
                table_hbm.at[idx_v.at[0]], bufs[b], g_sems[b]
            ).wait()

        def o_start(h, b):
            pltpu.async_copy(
                bufs[b],
                out_hbm.at[pl.ds(h * batch + col0, _CHUNK)],
                o_sems[b],
            )

        def o_wait(b):
            pltpu.make_async_copy(
                bufs[b], out_hbm.at[pl.ds(col0, _CHUNK)], o_sems[b]
            ).wait()

        for b in range(_NBUF - 1):
            g_start(b, b)

        def outer(o, carry):
            for b in range(_NBUF):
                h = o * _NBUF + b
                g_wait(b)
                o_start(h, b)
                pb = (b - 1) % _NBUF
                nxt = h + _NBUF - 1
                if b == 0:
                    @pl.when(o > 0)
                    def _():
                        o_wait(pb)

                    g_start(nxt, pb)
                elif b <= 1:
                    o_wait(pb)
                    g_start(nxt, pb)
                else:
                    o_wait(pb)

                    @pl.when(o < n_outer - 1)
                    def _():
                        g_start(nxt, pb)

            return carry

        lax.fori_loop(0, n_outer, outer, 0)

        g_wait(0)
        o_start(n_chunks - 1, 0)

        o_wait(_NBUF - 1)
        o_wait(0)

    return gather_kernel


def kernel(ids, table):
    batch, hist = ids.shape
    ids_t = ids.T.astype(jnp.int32)
    out = _make_gather(batch, hist)(ids_t, table)
    return out.reshape(hist, batch, _D).transpose(1, 0, 2)


# speedup vs baseline: 1.0350x; 1.0059x over previous
import functools
import jax
import jax.numpy as jnp
from jax import lax
from jax.experimental import pallas as pl
from jax.experimental.pallas import tpu as pltpu, tpu_sc as plsc

_D = 128
_NW = 32
_CHUNK = 128
_NBUF = 7


def _make_gather(batch, hist):
    n_chunks = hist
    cols = batch // _NW
    assert cols == _CHUNK
    n_outer = (n_chunks - 1) // _NBUF
    assert n_outer * _NBUF + 1 == n_chunks
    mesh = plsc.VectorSubcoreMesh(core_axis_name="c", subcore_axis_name="s")

    scratch = (
        [pltpu.VMEM((hist, cols), jnp.int32)]
        + [pltpu.VMEM((_CHUNK, _D), jnp.float32) for _ in range(_NBUF)]
        + [pltpu.SemaphoreType.DMA for _ in range(2 * _NBUF + 2)]
    )

    @functools.partial(
        pl.kernel,
        out_type=jax.ShapeDtypeStruct((hist * batch, _D), jnp.float32),
        mesh=mesh,
        scratch_types=scratch,
    )
    def gather_kernel(ids_hbm, table_hbm, out_hbm, idx_v, *rest):
        bufs = rest[:_NBUF]
        g_sems = rest[_NBUF : 2 * _NBUF]
        o_sems = rest[2 * _NBUF : 3 * _NBUF]
        i_sem0, i_sem1 = rest[3 * _NBUF :]
        wid = lax.axis_index("s") * 2 + lax.axis_index("c")
        col0 = wid * cols
        head = pltpu.async_copy(
            ids_hbm.at[pl.ds(0, 8), pl.ds(col0, cols)],
            idx_v.at[pl.ds(0, 8)],
            i_sem0,
        )
        tail = pltpu.async_copy(
            ids_hbm.at[pl.ds(8, hist - 8), pl.ds(col0, cols)],
            idx_v.at[pl.ds(8, hist - 8)],
            i_sem1,
        )
        head.wait()

        def g_start(h, b):
            pltpu.async_copy(table_hbm.at[idx_v.at[h]], bufs[b], g_sems[b])

        def g_wait(b):
            pltpu.make_async_copy(
                table_hbm.at[idx_v.at[0]], bufs[b], g_sems[b]
            ).wait()

        def o_start(h, b):
            pltpu.async_copy(
                bufs[b],
                out_hbm.at[pl.ds(h * batch + col0, _CHUNK)],
                o_sems[b],
            )

        def o_wait(b):
            pltpu.make_async_copy(
                bufs[b], out_hbm.at[pl.ds(col0, _CHUNK)], o_sems[b]
            ).wait()

        for b in range(_NBUF - 1):
            g_start(b, b)
        tail.wait()

        def outer(o, carry):
            for b in range(_NBUF):
                h = o * _NBUF + b
                g_wait(b)
                o_start(h, b)
                pb = (b - 1) % _NBUF
                nxt = h + _NBUF - 1
                if b == 0:
                    @pl.when(o > 0)
                    def _():
                        o_wait(pb)

                    g_start(nxt, pb)
                elif b <= 1:
                    o_wait(pb)
                    g_start(nxt, pb)
                else:
                    o_wait(pb)

                    @pl.when(o < n_outer - 1)
                    def _():
                        g_start(nxt, pb)

            return carry

        lax.fori_loop(0, n_outer, outer, 0)

        g_wait(0)
        o_start(n_chunks - 1, 0)

        o_wait(_NBUF - 1)
        o_wait(0)

    return gather_kernel


def kernel(ids, table):
    batch, hist = ids.shape
    ids_t = ids.T.astype(jnp.int32)
    out = _make_gather(batch, hist)(ids_t, table)
    return out.reshape(hist, batch, _D).transpose(1, 0, 2)
